# baseline (speedup 1.0000x reference)
import functools

import jax
import jax.numpy as jnp
from jax import lax
from jax.experimental import pallas as pl
from jax.experimental.pallas import tpu as pltpu
from jax.experimental.pallas import tpu_sc as plsc

S = 8192
D = 2048
NC = 2
NS = 16
L = 16
NW = NC * NS
CHUNK = S // NW
NB = CHUNK // L
NBUF = 3


def _body(inputs_hbm, mask_hbm, source_hbm, out_hbm,
          mask_v, buf0, buf1, buf2, mlist_src, mlist_pos, ulist_gidx,
          ulist_pos, sem_g0, sem_g1, sem_g2, sem_s0, sem_s1, sem_s2):
    wid = lax.axis_index("s") * NC + lax.axis_index("c")
    base = wid * CHUNK

    pltpu.sync_copy(mask_hbm, mask_v)

    iota = lax.iota(jnp.int32, L)
    zero16 = jnp.zeros((L,), jnp.int32)
    off_m = zero16
    off_u = zero16
    for j in range(NB):
        v = mask_v[pl.ds(base + j * L, L)]
        m = v > 0
        um = jnp.logical_not(m)
        cs = plsc.cumsum(v)
        pcnt = plsc.all_reduce_population_count(m)
        src_idx = off_m + cs - 1
        p = base + j * L + iota
        dest = off_m + cs - 1
        plsc.store_scatter(mlist_src, [dest >> 4, dest & 15], src_idx, mask=m)
        plsc.store_scatter(mlist_pos, [dest >> 4, dest & 15], p, mask=m)
        cs_u = plsc.cumsum(1 - v)
        dest_u = off_u + cs_u - 1
        plsc.store_scatter(ulist_gidx, [dest_u >> 4, dest_u & 15], p, mask=um)
        plsc.store_scatter(ulist_pos, [dest_u >> 4, dest_u & 15], p, mask=um)
        off_m = off_m + pcnt
        off_u = off_u + (L - pcnt)

    nm = jnp.max(off_m, axis=0)
    nu = CHUNK - nm

    def tail_fix(lst_a, lst_b, n):
        @pl.when(lax.rem(n, L) != 0)
        def _():
            full = n >> 4
            tsel = lax.rem(full * L + iota, jnp.full((L,), n, jnp.int32))
            row = tsel >> 4
            col = tsel & 15
            full_b = jnp.full((L,), full, jnp.int32)
            plsc.store_scatter(lst_a, [full_b, iota],
                               plsc.load_gather(lst_a, [row, col]))
            plsc.store_scatter(lst_b, [full_b, iota],
                               plsc.load_gather(lst_b, [row, col]))

    tail_fix(ulist_gidx, ulist_pos, nu)

    n_mb = (nm + (L - 1)) >> 4
    n_ub = (nu + (L - 1)) >> 4

    bufs = (buf0, buf1, buf2)
    gsems = (sem_g0, sem_g1, sem_g2)
    ssems = (sem_s0, sem_s1, sem_s2)

    @pl.when(n_ub > 0)
    def _():
        pltpu.async_copy(inputs_hbm.at[ulist_gidx.at[0]], buf0, sem_g0)

    @pl.when(n_ub > 1)
    def _():
        pltpu.async_copy(inputs_hbm.at[ulist_gidx.at[1]], buf1, sem_g1)

    def pf_body(j, acc):
        a = acc + mask_v[pl.ds(j * 4 * L, L)]
        a = a + mask_v[pl.ds((j * 4 + 1) * L, L)]
        a = a + mask_v[pl.ds((j * 4 + 2) * L, L)]
        return a + mask_v[pl.ds((j * 4 + 3) * L, L)]

    acc = lax.fori_loop(0, wid * (NB // 4), pf_body,
                        jnp.zeros((L,), jnp.int32))
    prefix = jnp.full((L,), jnp.sum(acc, axis=0), jnp.int32)
    for j in range(NB):
        @pl.when(j < n_mb)
        def _():
            mlist_src[j, :] = mlist_src[j, :] + prefix
    tail_fix(mlist_src, mlist_pos, nm)

    def stream(table_hbm, gidx, pos, n, primed=False):
        def gat(b, k):
            pltpu.async_copy(table_hbm.at[gidx.at[b]], bufs[k], gsems[k])

        def wgat(b, k):
            pltpu.make_async_copy(table_hbm.at[gidx.at[b]], bufs[k],
                                  gsems[k]).wait()

        def sct(b, k):
            pltpu.async_copy(bufs[k], out_hbm.at[pos.at[b]], ssems[k])

        def wsct(b, k):
            pltpu.make_async_copy(bufs[k], out_hbm.at[pos.at[b]],
                                  ssems[k]).wait()

        if not primed:
            @pl.when(n > 0)
            def _():
                gat(0, 0)

            @pl.when(n > 1)
            def _():
                gat(1, 1)

        def loop_body(t, _):
            for k in range(NBUF):
                b = NBUF * t + k

                @pl.when(b < n)
                def _():
                    @pl.when(b >= 1)
                    def _():
                        wsct(b - 1, (k + NBUF - 1) % NBUF)

                    @pl.when(b + 2 < n)
                    def _():
                        gat(b + 2, (k + 2) % NBUF)
                    wgat(b, k)
                    sct(b, k)
            return 0

        lax.fori_loop(0, (n + NBUF - 1) // NBUF, loop_body, 0)
        for k in range(NBUF):
            @pl.when((n > 0) & (lax.rem(n - 1, NBUF) == k))
            def _():
                wsct(n - 1, k)

    stream(inputs_hbm, ulist_gidx, ulist_pos, n_ub, primed=True)
    stream(source_hbm, mlist_src, mlist_pos, n_mb)


@functools.partial(
    pl.kernel,
    out_type=jax.ShapeDtypeStruct((S, D), jnp.float32),
    mesh=plsc.VectorSubcoreMesh(core_axis_name="c", subcore_axis_name="s"),
    compiler_params=pltpu.CompilerParams(needs_layout_passes=False),
    scratch_types=[
        pltpu.VMEM((S,), jnp.int32),
        pltpu.VMEM((L, D), jnp.float32),
        pltpu.VMEM((L, D), jnp.float32),
        pltpu.VMEM((L, D), jnp.float32),
        pltpu.VMEM((NB, L), jnp.int32),
        pltpu.VMEM((NB, L), jnp.int32),
        pltpu.VMEM((NB, L), jnp.int32),
        pltpu.VMEM((NB, L), jnp.int32),
        pltpu.SemaphoreType.DMA,
        pltpu.SemaphoreType.DMA,
        pltpu.SemaphoreType.DMA,
        pltpu.SemaphoreType.DMA,
        pltpu.SemaphoreType.DMA,
        pltpu.SemaphoreType.DMA,
    ],
)
def _sc_masked_scatter(inputs_hbm, mask_hbm, source_hbm, out_hbm, *scratch):
    _body(inputs_hbm, mask_hbm, source_hbm, out_hbm, *scratch)


def kernel(inputs_embeds, mask_1d, source):
    mask_i32 = mask_1d.astype(jnp.int32)
    return _sc_masked_scatter(inputs_embeds, mask_i32, source)


# speedup vs baseline: 1.0230x; 1.0230x over previous
import functools

import jax
import jax.numpy as jnp
from jax import lax
from jax.experimental import pallas as pl
from jax.experimental.pallas import tpu as pltpu
from jax.experimental.pallas import tpu_sc as plsc

S = 8192
D = 2048
NC = 2
NS = 16
L = 16
NW = NC * NS
CHUNK = S // NW
NB = CHUNK // L
NBUF = 3


def _body(inputs_hbm, mask_hbm, source_hbm, out_hbm,
          mask_v, buf0, buf1, buf2, mlist_src, mlist_pos, ulist_gidx,
          ulist_pos, sem_g0, sem_g1, sem_g2, sem_s0, sem_s1, sem_s2):
    wid = lax.axis_index("s") * NC + lax.axis_index("c")
    base = wid * CHUNK

    pltpu.sync_copy(mask_hbm, mask_v)

    iota = lax.iota(jnp.int32, L)
    zero16 = jnp.zeros((L,), jnp.int32)
    off_m = zero16
    off_u = zero16
    for j in range(NB):
        v = mask_v[pl.ds(base + j * L, L)]
        m = v > 0
        um = jnp.logical_not(m)
        cs = plsc.cumsum(v)
        pcnt = plsc.all_reduce_population_count(m)
        src_idx = off_m + cs - 1
        p = base + j * L + iota
        dest = off_m + cs - 1
        plsc.store_scatter(mlist_src, [dest >> 4, dest & 15], src_idx, mask=m)
        plsc.store_scatter(mlist_pos, [dest >> 4, dest & 15], p, mask=m)
        cs_u = plsc.cumsum(1 - v)
        dest_u = off_u + cs_u - 1
        plsc.store_scatter(ulist_gidx, [dest_u >> 4, dest_u & 15], p, mask=um)
        plsc.store_scatter(ulist_pos, [dest_u >> 4, dest_u & 15], p, mask=um)
        off_m = off_m + pcnt
        off_u = off_u + (L - pcnt)

    nm = jnp.max(off_m, axis=0)
    nu = CHUNK - nm

    def tail_fix(lst_a, lst_b, n):
        @pl.when(lax.rem(n, L) != 0)
        def _():
            full = n >> 4
            tsel = lax.rem(full * L + iota, jnp.full((L,), n, jnp.int32))
            row = tsel >> 4
            col = tsel & 15
            full_b = jnp.full((L,), full, jnp.int32)
            plsc.store_scatter(lst_a, [full_b, iota],
                               plsc.load_gather(lst_a, [row, col]))
            plsc.store_scatter(lst_b, [full_b, iota],
                               plsc.load_gather(lst_b, [row, col]))

    tail_fix(ulist_gidx, ulist_pos, nu)

    n_mb = (nm + (L - 1)) >> 4
    n_ub = (nu + (L - 1)) >> 4

    bufs = (buf0, buf1, buf2)
    gsems = (sem_g0, sem_g1, sem_g2)
    ssems = (sem_s0, sem_s1, sem_s2)

    nt = n_ub + n_mb

    def gat(g, k):
        @pl.when(g < n_ub)
        def _():
            pltpu.async_copy(inputs_hbm.at[ulist_gidx.at[g]], bufs[k],
                             gsems[k])

        @pl.when(g >= n_ub)
        def _():
            pltpu.async_copy(source_hbm.at[mlist_src.at[g - n_ub]], bufs[k],
                             gsems[k])

    def wgat(k):
        pltpu.make_async_copy(inputs_hbm.at[ulist_gidx.at[0]], bufs[k],
                              gsems[k]).wait()

    def sct(g, k):
        @pl.when(g < n_ub)
        def _():
            pltpu.async_copy(bufs[k], out_hbm.at[ulist_pos.at[g]], ssems[k])

        @pl.when(g >= n_ub)
        def _():
            pltpu.async_copy(bufs[k], out_hbm.at[mlist_pos.at[g - n_ub]],
                             ssems[k])

    def wsct(k):
        pltpu.make_async_copy(bufs[k], out_hbm.at[ulist_pos.at[0]],
                              ssems[k]).wait()

    @pl.when(n_ub > 0)
    def _():
        gat(0, 0)

    @pl.when(n_ub > 1)
    def _():
        gat(1, 1)

    def pf_body(j, acc):
        a = acc + mask_v[pl.ds(j * 4 * L, L)]
        a = a + mask_v[pl.ds((j * 4 + 1) * L, L)]
        a = a + mask_v[pl.ds((j * 4 + 2) * L, L)]
        return a + mask_v[pl.ds((j * 4 + 3) * L, L)]

    acc = lax.fori_loop(0, wid * (NB // 4), pf_body,
                        jnp.zeros((L,), jnp.int32))
    prefix = jnp.full((L,), jnp.sum(acc, axis=0), jnp.int32)
    for j in range(NB):
        @pl.when(j < n_mb)
        def _():
            mlist_src[j, :] = mlist_src[j, :] + prefix
    tail_fix(mlist_src, mlist_pos, nm)

    @pl.when(n_ub == 0)
    def _():
        gat(0, 0)

    @pl.when((n_ub <= 1) & (nt > 1))
    def _():
        gat(1, 1)

    def loop_body(t, _):
        for k in range(NBUF):
            g = NBUF * t + k

            @pl.when(g < nt)
            def _():
                @pl.when(g >= 1)
                def _():
                    wsct((k + NBUF - 1) % NBUF)

                @pl.when(g + 2 < nt)
                def _():
                    gat(g + 2, (k + 2) % NBUF)
                wgat(k)
                sct(g, k)
        return 0

    lax.fori_loop(0, (nt + NBUF - 1) // NBUF, loop_body, 0)
    for k in range(NBUF):
        @pl.when(lax.rem(nt - 1, NBUF) == k)
        def _():
            wsct(k)


@functools.partial(
    pl.kernel,
    out_type=jax.ShapeDtypeStruct((S, D), jnp.float32),
    mesh=plsc.VectorSubcoreMesh(core_axis_name="c", subcore_axis_name="s"),
    compiler_params=pltpu.CompilerParams(needs_layout_passes=False),
    scratch_types=[
        pltpu.VMEM((S,), jnp.int32),
        pltpu.VMEM((L, D), jnp.float32),
        pltpu.VMEM((L, D), jnp.float32),
        pltpu.VMEM((L, D), jnp.float32),
        pltpu.VMEM((NB, L), jnp.int32),
        pltpu.VMEM((NB, L), jnp.int32),
        pltpu.VMEM((NB, L), jnp.int32),
        pltpu.VMEM((NB, L), jnp.int32),
        pltpu.SemaphoreType.DMA,
        pltpu.SemaphoreType.DMA,
        pltpu.SemaphoreType.DMA,
        pltpu.SemaphoreType.DMA,
        pltpu.SemaphoreType.DMA,
        pltpu.SemaphoreType.DMA,
    ],
)
def _sc_masked_scatter(inputs_hbm, mask_hbm, source_hbm, out_hbm, *scratch):
    _body(inputs_hbm, mask_hbm, source_hbm, out_hbm, *scratch)


def kernel(inputs_embeds, mask_1d, source):
    mask_i32 = mask_1d.astype(jnp.int32)
    return _sc_masked_scatter(inputs_embeds, mask_i32, source)


# speedup vs baseline: 1.0456x; 1.0222x over previous
import functools

import jax
import jax.numpy as jnp
from jax import lax
from jax.experimental import pallas as pl
from jax.experimental.pallas import tpu as pltpu
from jax.experimental.pallas import tpu_sc as plsc

S = 8192
D = 2048
NC = 2
NS = 16
L = 16
NW = NC * NS
CHUNK = S // NW
NV = CHUNK // L
B = 8
NLR = CHUNK // B
NBUF = 5


def _body(inputs_hbm, mask_hbm, source_hbm, out_hbm,
          mask_v, buf0, buf1, buf2, buf3, buf4,
          mlist_src, mlist_pos, ulist_gidx, ulist_pos,
          sem_g0, sem_g1, sem_g2, sem_g3, sem_g4,
          sem_s0, sem_s1, sem_s2, sem_s3, sem_s4):
    wid = lax.axis_index("s") * NC + lax.axis_index("c")
    base = wid * CHUNK

    pltpu.sync_copy(mask_hbm, mask_v)

    iota = lax.iota(jnp.int32, L)
    zero16 = jnp.zeros((L,), jnp.int32)

    def pf_body(j, acc):
        a = acc + mask_v[pl.ds(j * 4 * L, L)]
        a = a + mask_v[pl.ds((j * 4 + 1) * L, L)]
        a = a + mask_v[pl.ds((j * 4 + 2) * L, L)]
        return a + mask_v[pl.ds((j * 4 + 3) * L, L)]

    acc = lax.fori_loop(0, wid * (NV // 4), pf_body, zero16)
    carry = jnp.full((L,), jnp.sum(acc, axis=0), jnp.int32)

    off_m = zero16
    off_u = zero16
    for j in range(NV):
        v = mask_v[pl.ds(base + j * L, L)]
        m = v > 0
        um = jnp.logical_not(m)
        cs = plsc.cumsum(v)
        pcnt = plsc.all_reduce_population_count(m)
        src_idx = carry + cs - 1
        p = base + j * L + iota
        dest = off_m + cs - 1
        plsc.store_scatter(mlist_src, [dest >> 3, dest & 7], src_idx, mask=m)
        plsc.store_scatter(mlist_pos, [dest >> 3, dest & 7], p, mask=m)
        dest_u = off_u + plsc.cumsum(1 - v) - 1
        plsc.store_scatter(ulist_gidx, [dest_u >> 3, dest_u & 7], p, mask=um)
        plsc.store_scatter(ulist_pos, [dest_u >> 3, dest_u & 7], p, mask=um)
        off_m = off_m + pcnt
        off_u = off_u + (L - pcnt)
        carry = carry + pcnt

    nm = jnp.max(off_m, axis=0)
    nu = CHUNK - nm

    def tail_fix(lst_a, lst_b, n):
        @pl.when(lax.rem(n, B) != 0)
        def _():
            full = n >> 3
            tsel = lax.rem(full * B + iota, jnp.full((L,), n, jnp.int32))
            row = tsel >> 3
            col = tsel & 7
            lane_ok = iota < B
            full_b = jnp.full((L,), full, jnp.int32)
            plsc.store_scatter(lst_a, [full_b, iota],
                               plsc.load_gather(lst_a, [row, col]),
                               mask=lane_ok)
            plsc.store_scatter(lst_b, [full_b, iota],
                               plsc.load_gather(lst_b, [row, col]),
                               mask=lane_ok)

    tail_fix(ulist_gidx, ulist_pos, nu)
    tail_fix(mlist_src, mlist_pos, nm)

    n_mb = (nm + (B - 1)) >> 3
    n_ub = (nu + (B - 1)) >> 3
    nt = n_ub + n_mb

    bufs = (buf0, buf1, buf2, buf3, buf4)
    gsems = (sem_g0, sem_g1, sem_g2, sem_g3, sem_g4)
    ssems = (sem_s0, sem_s1, sem_s2, sem_s3, sem_s4)

    def gat(g, k):
        @pl.when(g < n_ub)
        def _():
            pltpu.async_copy(inputs_hbm.at[ulist_gidx.at[g]], bufs[k],
                             gsems[k])

        @pl.when(g >= n_ub)
        def _():
            pltpu.async_copy(source_hbm.at[mlist_src.at[g - n_ub]], bufs[k],
                             gsems[k])

    def wgat(k):
        pltpu.make_async_copy(inputs_hbm.at[ulist_gidx.at[0]], bufs[k],
                              gsems[k]).wait()

    def sct(g, k):
        @pl.when(g < n_ub)
        def _():
            pltpu.async_copy(bufs[k], out_hbm.at[ulist_pos.at[g]], ssems[k])

        @pl.when(g >= n_ub)
        def _():
            pltpu.async_copy(bufs[k], out_hbm.at[mlist_pos.at[g - n_ub]],
                             ssems[k])

    def wsct(k):
        pltpu.make_async_copy(bufs[k], out_hbm.at[ulist_pos.at[0]],
                              ssems[k]).wait()

    for k in range(NBUF - 1):
        gat(k, k)

    def loop_body(t, _):
        for k in range(NBUF):
            g = NBUF * t + k

            @pl.when(g < nt)
            def _():
                @pl.when(g >= 1)
                def _():
                    wsct((k + NBUF - 1) % NBUF)

                @pl.when(g + NBUF - 1 < nt)
                def _():
                    gat(g + NBUF - 1, (k + NBUF - 1) % NBUF)
                wgat(k)
                sct(g, k)
        return 0

    lax.fori_loop(0, (nt + NBUF - 1) // NBUF, loop_body, 0)
    for k in range(NBUF):
        @pl.when(lax.rem(nt - 1, NBUF) == k)
        def _():
            wsct(k)


@functools.partial(
    pl.kernel,
    out_type=jax.ShapeDtypeStruct((S, D), jnp.float32),
    mesh=plsc.VectorSubcoreMesh(core_axis_name="c", subcore_axis_name="s"),
    compiler_params=pltpu.CompilerParams(needs_layout_passes=False),
    scratch_types=[
        pltpu.VMEM((S,), jnp.int32),
        pltpu.VMEM((B, D), jnp.float32),
        pltpu.VMEM((B, D), jnp.float32),
        pltpu.VMEM((B, D), jnp.float32),
        pltpu.VMEM((B, D), jnp.float32),
        pltpu.VMEM((B, D), jnp.float32),
        pltpu.VMEM((NLR, B), jnp.int32),
        pltpu.VMEM((NLR, B), jnp.int32),
        pltpu.VMEM((NLR, B), jnp.int32),
        pltpu.VMEM((NLR, B), jnp.int32),
        pltpu.SemaphoreType.DMA,
        pltpu.SemaphoreType.DMA,
        pltpu.SemaphoreType.DMA,
        pltpu.SemaphoreType.DMA,
        pltpu.SemaphoreType.DMA,
        pltpu.SemaphoreType.DMA,
        pltpu.SemaphoreType.DMA,
        pltpu.SemaphoreType.DMA,
        pltpu.SemaphoreType.DMA,
        pltpu.SemaphoreType.DMA,
    ],
)
def _sc_masked_scatter(inputs_hbm, mask_hbm, source_hbm, out_hbm, *scratch):
    _body(inputs_hbm, mask_hbm, source_hbm, out_hbm, *scratch)


def kernel(inputs_embeds, mask_1d, source):
    mask_i32 = mask_1d.astype(jnp.int32)
    return _sc_masked_scatter(inputs_embeds, mask_i32, source)
